# speedup vs baseline: 120.2350x; 120.2350x over previous
import jax
import jax.numpy as jnp
from jax.experimental import pallas as pl
from jax.experimental.pallas import tpu as pltpu

_V, _B, _C, _H, _W = 4, 8, 5, 512, 512
_HW = _H * _W
_Q = 85.0
_MIN_AREA = 0.01
_IGNORE = -1
_RH = 128
_NJ = _H // _RH
_RH2 = 256
_NJ2 = _H // _RH2


def _count_body(pred_ref, out_ref, s_ref):
    j = pl.program_id(1)
    x0 = pred_ref[0]
    x1 = pred_ref[1]
    x2 = pred_ref[2]
    x3 = pred_ref[3]
    x4 = pred_ref[4]
    p1 = x0
    p2 = jnp.maximum(p1, x1)
    p3 = jnp.maximum(p2, x2)
    p4 = jnp.maximum(p3, x3)
    s4 = x4
    s3 = jnp.maximum(x3, s4)
    s2 = jnp.maximum(x2, s3)
    s1 = jnp.maximum(x1, s2)
    g1 = jnp.sum((s1 > p1).astype(jnp.float32))
    g2 = jnp.sum((s2 > p2).astype(jnp.float32))
    g3 = jnp.sum((s3 > p3).astype(jnp.float32))
    g4 = jnp.sum((s4 > p4).astype(jnp.float32))

    @pl.when(j == 0)
    def _init():
        s_ref[0] = g1
        s_ref[1] = g2
        s_ref[2] = g3
        s_ref[3] = g4

    @pl.when(j > 0)
    def _acc():
        s_ref[0] += g1
        s_ref[1] += g2
        s_ref[2] += g3
        s_ref[3] += g4

    @pl.when(j == _NJ - 1)
    def _emit():
        sub = jax.lax.broadcasted_iota(jnp.int32, (8, 128), 0)
        lane = jax.lax.broadcasted_iota(jnp.int32, (8, 128), 1)
        vec = jnp.zeros((8, 128), jnp.float32)
        for k in range(4):
            vec = jnp.where((sub == 0) & (lane == k), s_ref[k], vec)
        out_ref[0] = vec


def _rewrite_body(anom_ref, lab_ref, out_ref):
    b = pl.program_id(0)
    lab = lab_ref[0]
    safe = jnp.clip(lab, 0, _C - 1)
    a0 = anom_ref[b, 0]
    a1 = anom_ref[b, 1]
    a2 = anom_ref[b, 2]
    a3 = anom_ref[b, 3]
    a4 = anom_ref[b, 4]
    af = jnp.where(
        safe == 0, a0,
        jnp.where(safe == 1, a1,
                  jnp.where(safe == 2, a2,
                            jnp.where(safe == 3, a3, a4))))
    out_ref[0] = jnp.where((lab != _IGNORE) & (af > 0), _IGNORE, lab)


def _quantile_thr(scores, mask):
    q = _Q / 100.0
    flat = jnp.where(mask, scores, jnp.inf).reshape(-1)
    s = jnp.sort(flat)
    n = jnp.sum(mask)
    pos = q * jnp.maximum(n - 1, 0).astype(jnp.float32)
    lo = jnp.floor(pos).astype(jnp.int32)
    hi = jnp.ceil(pos).astype(jnp.int32)
    frac = pos - lo.astype(jnp.float32)
    val = s[lo] * (1.0 - frac) + s[hi] * frac
    return jnp.where(n > 0, val, jnp.inf)


def kernel(predictions_list, pseudo_labels_to_modify):
    preds = predictions_list.reshape(_V * _B * _C, _H, _W)
    counts_s = pl.pallas_call(
        _count_body,
        grid=(_V * _B, _NJ),
        in_specs=[pl.BlockSpec((_C, _RH, _W), lambda i, j: (i, j, 0))],
        out_specs=pl.BlockSpec((1, 8, 128), lambda i, j: (i, 0, 0)),
        out_shape=jax.ShapeDtypeStruct((_V * _B, 8, 128), jnp.float32),
        scratch_shapes=[pltpu.SMEM((4,), jnp.float32)],
    )(preds)
    s = counts_s[:, 0, :4].reshape(_V, _B, 4)
    c0 = _HW - s[..., 0]
    c1 = s[..., 0] - s[..., 1]
    c2 = s[..., 1] - s[..., 2]
    c3 = s[..., 2] - s[..., 3]
    c4 = s[..., 3]
    stacked = jnp.stack([c0, c1, c2, c3, c4], axis=2)
    stacked = stacked.transpose(1, 2, 0)
    mean_c = jnp.mean(stacked, axis=2)
    std_c = jnp.std(stacked, axis=2, ddof=1)
    scores = std_c / (mean_c + 1e-08)
    scores = jnp.where(mean_c == 0, 0.0, scores)
    sig = (mean_c / _HW) > _MIN_AREA
    sig = sig.at[:, 0].set(False)
    thr = _quantile_thr(scores, sig)
    is_anom = ((scores > thr) & sig).astype(jnp.int32)
    final = pl.pallas_call(
        _rewrite_body,
        grid=(_B, _NJ2),
        in_specs=[
            pl.BlockSpec(memory_space=pltpu.SMEM),
            pl.BlockSpec((1, _RH2, _W), lambda b, j: (b, j, 0)),
        ],
        out_specs=pl.BlockSpec((1, _RH2, _W), lambda b, j: (b, j, 0)),
        out_shape=jax.ShapeDtypeStruct((_B, _H, _W), jnp.int32),
    )(is_anom, pseudo_labels_to_modify)
    return final


# speedup vs baseline: 201.3654x; 1.6748x over previous
import jax
import jax.numpy as jnp
from jax.experimental import pallas as pl
from jax.experimental.pallas import tpu as pltpu

_V, _B, _C, _H, _W = 4, 8, 5, 512, 512
_HW = _H * _W
_Q = 85.0
_MIN_AREA = 0.01
_IGNORE = -1
_RH = 512
_NJ = _H // _RH
_RH2 = 256
_NJ2 = _H // _RH2


def _count_body(pred_ref, out_ref, s_ref):
    j = pl.program_id(1)
    x0 = pred_ref[0]
    x1 = pred_ref[1]
    x2 = pred_ref[2]
    x3 = pred_ref[3]
    x4 = pred_ref[4]
    p1 = x0
    p2 = jnp.maximum(p1, x1)
    p3 = jnp.maximum(p2, x2)
    p4 = jnp.maximum(p3, x3)
    s4 = x4
    s3 = jnp.maximum(x3, s4)
    s2 = jnp.maximum(x2, s3)
    s1 = jnp.maximum(x1, s2)
    g1 = jnp.sum((s1 > p1).astype(jnp.float32))
    g2 = jnp.sum((s2 > p2).astype(jnp.float32))
    g3 = jnp.sum((s3 > p3).astype(jnp.float32))
    g4 = jnp.sum((s4 > p4).astype(jnp.float32))

    @pl.when(j == 0)
    def _init():
        s_ref[0] = g1
        s_ref[1] = g2
        s_ref[2] = g3
        s_ref[3] = g4

    @pl.when(j > 0)
    def _acc():
        s_ref[0] += g1
        s_ref[1] += g2
        s_ref[2] += g3
        s_ref[3] += g4

    @pl.when(j == _NJ - 1)
    def _emit():
        sub = jax.lax.broadcasted_iota(jnp.int32, (8, 128), 0)
        lane = jax.lax.broadcasted_iota(jnp.int32, (8, 128), 1)
        vec = jnp.zeros((8, 128), jnp.float32)
        for k in range(4):
            vec = jnp.where((sub == 0) & (lane == k), s_ref[k], vec)
        out_ref[0] = vec


def _rewrite_body(anom_ref, lab_ref, out_ref):
    b = pl.program_id(0)
    lab = lab_ref[0]
    safe = jnp.clip(lab, 0, _C - 1)
    a0 = anom_ref[b, 0]
    a1 = anom_ref[b, 1]
    a2 = anom_ref[b, 2]
    a3 = anom_ref[b, 3]
    a4 = anom_ref[b, 4]
    af = jnp.where(
        safe == 0, a0,
        jnp.where(safe == 1, a1,
                  jnp.where(safe == 2, a2,
                            jnp.where(safe == 3, a3, a4))))
    out_ref[0] = jnp.where((lab != _IGNORE) & (af > 0), _IGNORE, lab)


def _quantile_thr(scores, mask):
    q = _Q / 100.0
    flat = jnp.where(mask, scores, jnp.inf).reshape(-1)
    s = jnp.sort(flat)
    n = jnp.sum(mask)
    pos = q * jnp.maximum(n - 1, 0).astype(jnp.float32)
    lo = jnp.floor(pos).astype(jnp.int32)
    hi = jnp.ceil(pos).astype(jnp.int32)
    frac = pos - lo.astype(jnp.float32)
    val = s[lo] * (1.0 - frac) + s[hi] * frac
    return jnp.where(n > 0, val, jnp.inf)


def kernel(predictions_list, pseudo_labels_to_modify):
    preds = predictions_list.reshape(_V * _B * _C, _H, _W)
    counts_s = pl.pallas_call(
        _count_body,
        grid=(_V * _B, _NJ),
        in_specs=[pl.BlockSpec((_C, _RH, _W), lambda i, j: (i, j, 0))],
        out_specs=pl.BlockSpec((1, 8, 128), lambda i, j: (i, 0, 0)),
        out_shape=jax.ShapeDtypeStruct((_V * _B, 8, 128), jnp.float32),
        scratch_shapes=[pltpu.SMEM((4,), jnp.float32)],
    )(preds)
    s = counts_s[:, 0, :4].reshape(_V, _B, 4)
    c0 = _HW - s[..., 0]
    c1 = s[..., 0] - s[..., 1]
    c2 = s[..., 1] - s[..., 2]
    c3 = s[..., 2] - s[..., 3]
    c4 = s[..., 3]
    stacked = jnp.stack([c0, c1, c2, c3, c4], axis=2)
    stacked = stacked.transpose(1, 2, 0)
    mean_c = jnp.mean(stacked, axis=2)
    std_c = jnp.std(stacked, axis=2, ddof=1)
    scores = std_c / (mean_c + 1e-08)
    scores = jnp.where(mean_c == 0, 0.0, scores)
    sig = (mean_c / _HW) > _MIN_AREA
    sig = sig.at[:, 0].set(False)
    thr = _quantile_thr(scores, sig)
    is_anom = ((scores > thr) & sig).astype(jnp.int32)
    final = pl.pallas_call(
        _rewrite_body,
        grid=(_B, _NJ2),
        in_specs=[
            pl.BlockSpec(memory_space=pltpu.SMEM),
            pl.BlockSpec((1, _RH2, _W), lambda b, j: (b, j, 0)),
        ],
        out_specs=pl.BlockSpec((1, _RH2, _W), lambda b, j: (b, j, 0)),
        out_shape=jax.ShapeDtypeStruct((_B, _H, _W), jnp.int32),
    )(is_anom, pseudo_labels_to_modify)
    return final


# speedup vs baseline: 219.8397x; 1.0917x over previous
import jax
import jax.numpy as jnp
from jax.experimental import pallas as pl
from jax.experimental.pallas import tpu as pltpu

_V, _B, _C, _H, _W = 4, 8, 5, 512, 512
_HW = _H * _W
_Q = 85.0
_MIN_AREA = 0.01
_IGNORE = -1
_RH = 512
_NJ = _H // _RH
_RH2 = 256
_NJ2 = _H // _RH2


_G = 2


def _count_body(pred_ref, out_ref):
    sub = jax.lax.broadcasted_iota(jnp.int32, (8, 128), 0)
    lane = jax.lax.broadcasted_iota(jnp.int32, (8, 128), 1)
    for g in range(_G):
        x0 = pred_ref[5 * g + 0]
        x1 = pred_ref[5 * g + 1]
        x2 = pred_ref[5 * g + 2]
        x3 = pred_ref[5 * g + 3]
        x4 = pred_ref[5 * g + 4]
        p2 = jnp.maximum(x0, x1)
        p3 = jnp.maximum(p2, x2)
        p4 = jnp.maximum(p3, x3)
        s3 = jnp.maximum(x3, x4)
        s2 = jnp.maximum(x2, s3)
        s1 = jnp.maximum(x1, s2)
        g1 = jnp.sum((s1 > x0).astype(jnp.float32))
        g2 = jnp.sum((s2 > p2).astype(jnp.float32))
        g3 = jnp.sum((s3 > p3).astype(jnp.float32))
        g4 = jnp.sum((x4 > p4).astype(jnp.float32))
        vec = jnp.zeros((8, 128), jnp.float32)
        for k, gk in enumerate((g1, g2, g3, g4)):
            vec = jnp.where((sub == 0) & (lane == k), gk, vec)
        out_ref[g] = vec


def _rewrite_body(anom_ref, lab_ref, out_ref):
    b = pl.program_id(0)
    lab = lab_ref[0]
    safe = jnp.clip(lab, 0, _C - 1)
    a0 = anom_ref[b, 0]
    a1 = anom_ref[b, 1]
    a2 = anom_ref[b, 2]
    a3 = anom_ref[b, 3]
    a4 = anom_ref[b, 4]
    af = jnp.where(
        safe == 0, a0,
        jnp.where(safe == 1, a1,
                  jnp.where(safe == 2, a2,
                            jnp.where(safe == 3, a3, a4))))
    out_ref[0] = jnp.where((lab != _IGNORE) & (af > 0), _IGNORE, lab)


def _quantile_thr(scores, mask):
    q = _Q / 100.0
    flat = jnp.where(mask, scores, jnp.inf).reshape(-1)
    s = jnp.sort(flat)
    n = jnp.sum(mask)
    pos = q * jnp.maximum(n - 1, 0).astype(jnp.float32)
    lo = jnp.floor(pos).astype(jnp.int32)
    hi = jnp.ceil(pos).astype(jnp.int32)
    frac = pos - lo.astype(jnp.float32)
    val = s[lo] * (1.0 - frac) + s[hi] * frac
    return jnp.where(n > 0, val, jnp.inf)


def kernel(predictions_list, pseudo_labels_to_modify):
    preds = predictions_list.reshape(_V * _B * _C, _H, _W)
    counts_s = pl.pallas_call(
        _count_body,
        grid=(_V * _B // _G,),
        in_specs=[pl.BlockSpec((_C * _G, _H, _W), lambda i: (i, 0, 0))],
        out_specs=pl.BlockSpec((_G, 8, 128), lambda i: (i, 0, 0)),
        out_shape=jax.ShapeDtypeStruct((_V * _B, 8, 128), jnp.float32),
    )(preds)
    s = counts_s[:, 0, :4].reshape(_V, _B, 4)
    c0 = _HW - s[..., 0]
    c1 = s[..., 0] - s[..., 1]
    c2 = s[..., 1] - s[..., 2]
    c3 = s[..., 2] - s[..., 3]
    c4 = s[..., 3]
    stacked = jnp.stack([c0, c1, c2, c3, c4], axis=2)
    stacked = stacked.transpose(1, 2, 0)
    mean_c = jnp.mean(stacked, axis=2)
    std_c = jnp.std(stacked, axis=2, ddof=1)
    scores = std_c / (mean_c + 1e-08)
    scores = jnp.where(mean_c == 0, 0.0, scores)
    sig = (mean_c / _HW) > _MIN_AREA
    sig = sig.at[:, 0].set(False)
    thr = _quantile_thr(scores, sig)
    is_anom = ((scores > thr) & sig).astype(jnp.int32)
    final = pl.pallas_call(
        _rewrite_body,
        grid=(_B, _NJ2),
        in_specs=[
            pl.BlockSpec(memory_space=pltpu.SMEM),
            pl.BlockSpec((1, _RH2, _W), lambda b, j: (b, j, 0)),
        ],
        out_specs=pl.BlockSpec((1, _RH2, _W), lambda b, j: (b, j, 0)),
        out_shape=jax.ShapeDtypeStruct((_B, _H, _W), jnp.int32),
    )(is_anom, pseudo_labels_to_modify)
    return final
